# baseline (speedup 1.0000x reference)
import functools

import jax
import jax.numpy as jnp
from jax import lax
from jax.experimental import pallas as pl
from jax.experimental.pallas import tpu as pltpu
from jax.experimental.pallas import tpu_sc as plsc

VOCAB_SIZE = 1000000
EMB_D = 32
N_CLS = 100
BATCH_N = 4096
SEQ_N = 200

NUM_CORES = 2
NUM_SUBCORES = 16
NUM_WORKERS = NUM_CORES * NUM_SUBCORES
B_PER_W = BATCH_N // NUM_WORKERS
N_CHUNK = 2
CHUNK = SEQ_N // N_CHUNK
LANES = 16

WIN = 512
N_WIN = VOCAB_SIZE // WIN
ROWS_PER_WIN = WIN // 4
TAIL = VOCAB_SIZE - N_WIN * WIN


def _mesh():
    return plsc.VectorSubcoreMesh(
        core_axis_name="c", subcore_axis_name="s",
        num_cores=NUM_CORES, num_subcores=NUM_SUBCORES)


def _sc_transpose(tab_t):
    n_steps = -(-N_WIN // NUM_WORKERS)
    NG = EMB_D // 8

    @functools.partial(
        pl.kernel,
        out_type=jax.ShapeDtypeStruct((VOCAB_SIZE // 4, 128), jnp.float32),
        mesh=_mesh(),
        compiler_params=pltpu.CompilerParams(
            use_tc_tiling_on_sc=True, needs_layout_passes=False),
        scratch_types=[
            pltpu.VMEM((NG, 8, WIN), jnp.float32),
            pltpu.VMEM((NG, 8, WIN), jnp.float32),
            pltpu.VMEM((ROWS_PER_WIN, 128), jnp.float32),
            pltpu.VMEM((ROWS_PER_WIN, 128), jnp.float32),
            pltpu.SemaphoreType.DMA,
            pltpu.SemaphoreType.DMA,
            pltpu.SemaphoreType.DMA,
            pltpu.SemaphoreType.DMA,
        ],
    )
    def transp(tab_hbm, out_hbm, inb0, inb1, outb0, outb1, si0, si1, so0, so1):
        wid = lax.axis_index("s") * NUM_CORES + lax.axis_index("c")
        inbufs = (inb0, inb1)
        outbufs = (outb0, outb1)
        sis = (si0, si1)
        sos = (so0, so1)

        def win_of(k):
            return wid + k * NUM_WORKERS

        def fire_in(k, buf):
            w = win_of(k)

            @pl.when(w < N_WIN)
            def _():
                for g in range(NG):
                    pltpu.async_copy(
                        tab_hbm.at[pl.ds(8 * g, 8), pl.ds(w * WIN, WIN)],
                        inbufs[buf].at[g], sis[buf])

        def wait_in(k, buf):
            w = win_of(k)
            for g in range(NG):
                pltpu.make_async_copy(
                    tab_hbm.at[pl.ds(8 * g, 8), pl.ds(w * WIN, WIN)],
                    inbufs[buf].at[g], sis[buf]).wait()

        def fire_out(k, buf):
            w = win_of(k)
            pltpu.async_copy(
                outbufs[buf], out_hbm.at[pl.ds(w * ROWS_PER_WIN,
                                               ROWS_PER_WIN)], sos[buf])

        def wait_out(k, buf):
            w = win_of(k)
            pltpu.make_async_copy(
                outbufs[buf], out_hbm.at[pl.ds(w * ROWS_PER_WIN,
                                               ROWS_PER_WIN)], sos[buf]).wait()

        iota = lax.iota(jnp.int32, LANES)
        g_idx = [(16 * (h % 2) + iota) // 8 for h in range(2)]
        s_idx = iota % 8

        def shuffle(buf):
            src = inbufs[buf]
            dst = outbufs[buf]

            def body(j, carry):
                xs = []
                for h in range(8):
                    v_idx = jnp.full((LANES,), 4 * j + h // 2, jnp.int32)
                    xs.append(plsc.load_gather(
                        src, [g_idx[h % 2], s_idx, v_idx]))
                for h in range(8):
                    dst[j, pl.ds(16 * h, LANES)] = xs[h]
                return carry

            lax.fori_loop(0, ROWS_PER_WIN, body, 0)

        fire_in(0, 0)
        fire_in(1, 1)

        def step(k, carry):
            for buf in range(2):
                kk = 2 * k + buf

                @pl.when(kk < n_steps)
                def _():
                    @pl.when((kk >= 2) & (win_of(kk - 2) < N_WIN))
                    def _():
                        wait_out(kk - 2, buf)

                    @pl.when(win_of(kk) < N_WIN)
                    def _():
                        wait_in(kk, buf)
                        shuffle(buf)
                        fire_out(kk, buf)
                    fire_in(kk + 2, buf)
            return carry

        lax.fori_loop(0, (n_steps + 1) // 2, step, 0)
        for kk in (n_steps - 2, n_steps - 1):

            @pl.when(win_of(kk) < N_WIN)
            def _():
                wait_out(kk, kk % 2)


    return transp(tab_t)


def _sc_pool(idx3, table):
    @functools.partial(
        pl.kernel,
        out_type=jax.ShapeDtypeStruct((BATCH_N, EMB_D), jnp.float32),
        mesh=_mesh(),
        compiler_params=pltpu.CompilerParams(use_tc_tiling_on_sc=False),
        scratch_types=[
            pltpu.VMEM((B_PER_W, N_CHUNK, CHUNK), jnp.int32),
            pltpu.VMEM((2, N_CHUNK, CHUNK, EMB_D), jnp.float32),
            pltpu.VMEM((B_PER_W, EMB_D), jnp.float32),
            pltpu.SemaphoreType.DMA,
            pltpu.SemaphoreType.DMA,
        ],
    )
    def pool(idx_hbm, table_hbm, out_hbm, idx_v, rows_v, out_v, sem0, sem1):
        wid = lax.axis_index("s") * NUM_CORES + lax.axis_index("c")
        base = wid * B_PER_W
        pltpu.sync_copy(idx_hbm.at[pl.ds(base, B_PER_W)], idx_v)
        sems = (sem0, sem1)

        def fire(u, buf):
            @pl.when(u < B_PER_W)
            def _():
                for c in range(N_CHUNK):
                    pltpu.async_copy(
                        table_hbm.at[idx_v.at[u, c]], rows_v.at[buf, c],
                        sems[buf])

        def drain(u, buf):
            for c in range(N_CHUNK):
                pltpu.make_async_copy(
                    table_hbm.at[idx_v.at[u, c]], rows_v.at[buf, c],
                    sems[buf]).wait()

        def accum(u, buf):
            def body(s, carry):
                a0, a1 = carry
                for c in range(N_CHUNK):
                    a0 = a0 + rows_v[buf, c, s, pl.ds(0, LANES)]
                    a1 = a1 + rows_v[buf, c, s, pl.ds(LANES, LANES)]
                return a0, a1
            zero = jnp.zeros((LANES,), jnp.float32)
            a0, a1 = lax.fori_loop(0, CHUNK, body, (zero, zero))
            out_v[u, pl.ds(0, LANES)] = a0
            out_v[u, pl.ds(LANES, LANES)] = a1

        fire(0, 0)
        fire(1, 1)

        def outer(i, carry):
            g = 2 * i
            drain(g, 0)
            accum(g, 0)
            fire(g + 2, 0)
            drain(g + 1, 1)
            accum(g + 1, 1)
            fire(g + 3, 1)
            return carry

        lax.fori_loop(0, B_PER_W // 2, outer, 0)
        pltpu.sync_copy(out_v, out_hbm.at[pl.ds(base, B_PER_W)])

    return pool(idx3, table)


def _tc_linear(pooled, wt, b2):
    bm = 512

    def body(x_ref, w_ref, b_ref, o_ref):
        o_ref[...] = jnp.dot(
            x_ref[...], w_ref[...],
            precision=jax.lax.Precision.HIGHEST,
            preferred_element_type=jnp.float32) + b_ref[...]

    return pl.pallas_call(
        body,
        grid=(BATCH_N // bm,),
        in_specs=[
            pl.BlockSpec((bm, EMB_D), lambda i: (i, 0)),
            pl.BlockSpec((EMB_D, N_CLS), lambda i: (0, 0)),
            pl.BlockSpec((1, N_CLS), lambda i: (0, 0)),
        ],
        out_specs=pl.BlockSpec((bm, N_CLS), lambda i: (i, 0)),
        out_shape=jax.ShapeDtypeStruct((BATCH_N, N_CLS), jnp.float32),
    )(pooled, wt, b2)


def kernel(utteranceTokens, table, W, b):
    idx3 = utteranceTokens.astype(jnp.int32).reshape(BATCH_N, N_CHUNK, CHUNK)
    t_rm = _sc_transpose(table.T)
    tail_rows = table[N_WIN * WIN:].reshape(TAIL // 4, 4 * EMB_D)
    t_rm = lax.dynamic_update_slice(t_rm, tail_rows, (N_WIN * ROWS_PER_WIN, 0))
    tab_lin = t_rm.reshape(VOCAB_SIZE, EMB_D)
    pooled = _sc_pool(idx3, tab_lin)
    wt = (W.astype(jnp.float32) * (1.0 / SEQ_N)).T
    b2 = b.reshape(1, N_CLS)
    return _tc_linear(pooled, wt, b2)


# speedup vs baseline: 1.0494x; 1.0494x over previous
import functools

import jax
import jax.numpy as jnp
from jax import lax
from jax.experimental import pallas as pl
from jax.experimental.pallas import tpu as pltpu
from jax.experimental.pallas import tpu_sc as plsc

VOCAB_SIZE = 1000000
EMB_D = 32
N_CLS = 100
BATCH_N = 4096
SEQ_N = 200

NUM_CORES = 2
NUM_SUBCORES = 16
NUM_WORKERS = NUM_CORES * NUM_SUBCORES
B_PER_W = BATCH_N // NUM_WORKERS
N_CHUNK = 2
CHUNK = SEQ_N // N_CHUNK
LANES = 16

WIN = 512
N_WIN = VOCAB_SIZE // WIN
ROWS_PER_WIN = WIN // 4
TAIL = VOCAB_SIZE - N_WIN * WIN


def _mesh():
    return plsc.VectorSubcoreMesh(
        core_axis_name="c", subcore_axis_name="s",
        num_cores=NUM_CORES, num_subcores=NUM_SUBCORES)


def _sc_transpose(tab_t):
    n_steps = -(-N_WIN // NUM_WORKERS)
    NG = EMB_D // 8

    @functools.partial(
        pl.kernel,
        out_type=jax.ShapeDtypeStruct((VOCAB_SIZE // 4, 128), jnp.float32),
        mesh=_mesh(),
        compiler_params=pltpu.CompilerParams(
            use_tc_tiling_on_sc=True, needs_layout_passes=False),
        scratch_types=[
            pltpu.VMEM((NG, 8, WIN), jnp.float32),
            pltpu.VMEM((NG, 8, WIN), jnp.float32),
            pltpu.VMEM((ROWS_PER_WIN, 128), jnp.float32),
            pltpu.VMEM((ROWS_PER_WIN, 128), jnp.float32),
            pltpu.SemaphoreType.DMA,
            pltpu.SemaphoreType.DMA,
            pltpu.SemaphoreType.DMA,
            pltpu.SemaphoreType.DMA,
        ],
    )
    def transp(tab_hbm, out_hbm, inb0, inb1, outb0, outb1, si0, si1, so0, so1):
        wid = lax.axis_index("s") * NUM_CORES + lax.axis_index("c")
        inbufs = (inb0, inb1)
        outbufs = (outb0, outb1)
        sis = (si0, si1)
        sos = (so0, so1)

        def win_of(k):
            return wid + k * NUM_WORKERS

        def fire_in(k, buf):
            w = win_of(k)

            @pl.when(w < N_WIN)
            def _():
                for g in range(NG):
                    pltpu.async_copy(
                        tab_hbm.at[pl.ds(8 * g, 8), pl.ds(w * WIN, WIN)],
                        inbufs[buf].at[g], sis[buf])

        def wait_in(k, buf):
            w = win_of(k)
            for g in range(NG):
                pltpu.make_async_copy(
                    tab_hbm.at[pl.ds(8 * g, 8), pl.ds(w * WIN, WIN)],
                    inbufs[buf].at[g], sis[buf]).wait()

        def fire_out(k, buf):
            w = win_of(k)
            pltpu.async_copy(
                outbufs[buf], out_hbm.at[pl.ds(w * ROWS_PER_WIN,
                                               ROWS_PER_WIN)], sos[buf])

        def wait_out(k, buf):
            w = win_of(k)
            pltpu.make_async_copy(
                outbufs[buf], out_hbm.at[pl.ds(w * ROWS_PER_WIN,
                                               ROWS_PER_WIN)], sos[buf]).wait()

        iota = lax.iota(jnp.int32, LANES)
        row_pat = iota // 4
        col_base = 32 * (iota % 4)

        def shuffle(buf):
            src = inbufs[buf]
            dst = outbufs[buf]

            def body(h, carry):
                rows = row_pat + 4 * h
                for half in range(2):
                    ds_ = range(half * 16, half * 16 + 16)
                    xs = [src[d // 8, d % 8, pl.ds(16 * h, LANES)]
                          for d in ds_]
                    for x, d in zip(xs, ds_):
                        plsc.store_scatter(dst, [rows, col_base + d], x)
                return carry

            lax.fori_loop(0, WIN // LANES, body, 0)

        fire_in(0, 0)
        fire_in(1, 1)

        def step(k, carry):
            for buf in range(2):
                kk = 2 * k + buf

                @pl.when(kk < n_steps)
                def _():
                    @pl.when((kk >= 2) & (win_of(kk - 2) < N_WIN))
                    def _():
                        wait_out(kk - 2, buf)

                    @pl.when(win_of(kk) < N_WIN)
                    def _():
                        wait_in(kk, buf)
                        shuffle(buf)
                        fire_out(kk, buf)
                    fire_in(kk + 2, buf)
            return carry

        lax.fori_loop(0, (n_steps + 1) // 2, step, 0)
        for kk in (n_steps - 2, n_steps - 1):

            @pl.when(win_of(kk) < N_WIN)
            def _():
                wait_out(kk, kk % 2)


    return transp(tab_t)


def _sc_pool(idx3, table):
    @functools.partial(
        pl.kernel,
        out_type=jax.ShapeDtypeStruct((BATCH_N, EMB_D), jnp.float32),
        mesh=_mesh(),
        compiler_params=pltpu.CompilerParams(use_tc_tiling_on_sc=False),
        scratch_types=[
            pltpu.VMEM((B_PER_W, N_CHUNK, CHUNK), jnp.int32),
            pltpu.VMEM((2, N_CHUNK, CHUNK, EMB_D), jnp.float32),
            pltpu.VMEM((B_PER_W, EMB_D), jnp.float32),
            pltpu.SemaphoreType.DMA,
            pltpu.SemaphoreType.DMA,
        ],
    )
    def pool(idx_hbm, table_hbm, out_hbm, idx_v, rows_v, out_v, sem0, sem1):
        wid = lax.axis_index("s") * NUM_CORES + lax.axis_index("c")
        base = wid * B_PER_W
        pltpu.sync_copy(idx_hbm.at[pl.ds(base, B_PER_W)], idx_v)
        sems = (sem0, sem1)

        def fire(u, buf):
            @pl.when(u < B_PER_W)
            def _():
                for c in range(N_CHUNK):
                    pltpu.async_copy(
                        table_hbm.at[idx_v.at[u, c]], rows_v.at[buf, c],
                        sems[buf])

        def drain(u, buf):
            for c in range(N_CHUNK):
                pltpu.make_async_copy(
                    table_hbm.at[idx_v.at[u, c]], rows_v.at[buf, c],
                    sems[buf]).wait()

        def accum(u, buf):
            def body(i, carry):
                aE0, aE1, aO0, aO1 = carry
                xs = []
                for s_off in range(2):
                    for c in range(N_CHUNK):
                        for half in range(2):
                            xs.append(rows_v[buf, c, 2 * i + s_off,
                                             pl.ds(half * LANES, LANES)])
                aE0 = aE0 + xs[0] + xs[2]
                aE1 = aE1 + xs[1] + xs[3]
                aO0 = aO0 + xs[4] + xs[6]
                aO1 = aO1 + xs[5] + xs[7]
                return aE0, aE1, aO0, aO1
            zero = jnp.zeros((LANES,), jnp.float32)
            aE0, aE1, aO0, aO1 = lax.fori_loop(
                0, CHUNK // 2, body, (zero, zero, zero, zero))
            out_v[u, pl.ds(0, LANES)] = aE0 + aO0
            out_v[u, pl.ds(LANES, LANES)] = aE1 + aO1

        fire(0, 0)
        fire(1, 1)

        def outer(i, carry):
            g = 2 * i
            drain(g, 0)
            accum(g, 0)
            fire(g + 2, 0)
            drain(g + 1, 1)
            accum(g + 1, 1)
            fire(g + 3, 1)
            return carry

        lax.fori_loop(0, B_PER_W // 2, outer, 0)
        pltpu.sync_copy(out_v, out_hbm.at[pl.ds(base, B_PER_W)])

    return pool(idx3, table)


def _tc_linear(pooled, wt, b2):
    bm = 512

    def body(x_ref, w_ref, b_ref, o_ref):
        o_ref[...] = jnp.dot(
            x_ref[...], w_ref[...],
            precision=jax.lax.Precision.HIGHEST,
            preferred_element_type=jnp.float32) + b_ref[...]

    return pl.pallas_call(
        body,
        grid=(BATCH_N // bm,),
        in_specs=[
            pl.BlockSpec((bm, EMB_D), lambda i: (i, 0)),
            pl.BlockSpec((EMB_D, N_CLS), lambda i: (0, 0)),
            pl.BlockSpec((1, N_CLS), lambda i: (0, 0)),
        ],
        out_specs=pl.BlockSpec((bm, N_CLS), lambda i: (i, 0)),
        out_shape=jax.ShapeDtypeStruct((BATCH_N, N_CLS), jnp.float32),
    )(pooled, wt, b2)


def kernel(utteranceTokens, table, W, b):
    idx3 = utteranceTokens.astype(jnp.int32).reshape(BATCH_N, N_CHUNK, CHUNK)
    t_rm = _sc_transpose(table.T)
    tail_rows = table[N_WIN * WIN:].reshape(TAIL // 4, 4 * EMB_D)
    t_rm = lax.dynamic_update_slice(t_rm, tail_rows, (N_WIN * ROWS_PER_WIN, 0))
    tab_lin = t_rm.reshape(VOCAB_SIZE, EMB_D)
    pooled = _sc_pool(idx3, tab_lin)
    wt = (W.astype(jnp.float32) * (1.0 / SEQ_N)).T
    b2 = b.reshape(1, N_CLS)
    return _tc_linear(pooled, wt, b2)
